# baseline (speedup 1.0000x reference)
import functools

import jax
import jax.numpy as jnp
from jax import lax
from jax.experimental import pallas as pl
from jax.experimental.pallas import tpu as pltpu
from jax.experimental.pallas import tpu_sc as plsc

N = 10000
E = 320000
D = 128
ED = 16
H = 128
W = 144

BN = 1000
BE = 2000

NC = 2
NS = 16
CHK = 80
CH = E // (NC * NS * CHK)
EPT = CH * CHK
CH2 = E // (NS * CHK)


def _silu(v):
    return v * jax.nn.sigmoid(v)


WG = 256


def _prep_body(h_ref, xp_ref, xn_ref, wa_ref, wb_ref, ts_ref, td_ref):
    h = h_ref[...]
    z = jnp.zeros((h.shape[0], WG - D - 16), jnp.float32)
    ts_ref[...] = jnp.concatenate([jnp.dot(h, wa_ref[...],
                                           preferred_element_type=jnp.float32),
                                   xp_ref[...], z], axis=1)
    td_ref[...] = jnp.concatenate([jnp.dot(h, wb_ref[...],
                                           preferred_element_type=jnp.float32),
                                   xn_ref[...], z], axis=1)


def _prep(h, xp, xn, wa, wb):
    grid = (N // BN,)
    return pl.pallas_call(
        _prep_body,
        grid=grid,
        in_specs=[
            pl.BlockSpec((BN, D), lambda i: (i, 0)),
            pl.BlockSpec((BN, 16), lambda i: (i, 0)),
            pl.BlockSpec((BN, 16), lambda i: (i, 0)),
            pl.BlockSpec((D, D), lambda i: (0, 0)),
            pl.BlockSpec((D, D), lambda i: (0, 0)),
        ],
        out_specs=[
            pl.BlockSpec((BN, WG), lambda i: (i, 0)),
            pl.BlockSpec((BN, WG), lambda i: (i, 0)),
        ],
        out_shape=[
            jax.ShapeDtypeStruct((N, WG), jnp.float32),
            jax.ShapeDtypeStruct((N, WG), jnp.float32),
        ],
    )(h, xp, xn, wa, wb)


def _sc_mesh():
    return plsc.VectorSubcoreMesh(core_axis_name="c", subcore_axis_name="s")


def _gather(ta, tb, src5, dst5):
    @functools.partial(
        pl.kernel,
        out_type=[
            jax.ShapeDtypeStruct((E, D), jnp.float32),
            jax.ShapeDtypeStruct((E, 16), jnp.float32),
        ],
        mesh=_sc_mesh(),
        scratch_types=[
            pltpu.VMEM((1, CHK), jnp.int32),
            pltpu.VMEM((1, CHK), jnp.int32),
            pltpu.VMEM((1, CHK), jnp.int32),
            pltpu.VMEM((1, CHK), jnp.int32),
            pltpu.VMEM((CHK, WG), jnp.float32),
            pltpu.VMEM((CHK, WG), jnp.float32),
            pltpu.VMEM((CHK, WG), jnp.float32),
            pltpu.VMEM((CHK, WG), jnp.float32),
            pltpu.VMEM((CHK, D), jnp.float32),
            pltpu.VMEM((CHK, D), jnp.float32),
            pltpu.VMEM((CHK, 16), jnp.float32),
            pltpu.VMEM((CHK, 16), jnp.float32),
            pltpu.SemaphoreType.DMA,
            pltpu.SemaphoreType.DMA,
            pltpu.SemaphoreType.DMA,
            pltpu.SemaphoreType.DMA,
            pltpu.SemaphoreType.DMA,
            pltpu.SemaphoreType.DMA,
            pltpu.SemaphoreType.DMA,
            pltpu.SemaphoreType.DMA,
        ],
    )
    def k(ta_h, tb_h, src_h, dst_h, o_h, r_h,
          is0, is1, id0, id1,
          ba0, ba1, bb0, bb1, bo0, bo1, br0, br1,
          sa0, sa1, sb0, sb1, so0, so1, si0, si1):
        cid = lax.axis_index("c")
        sid = lax.axis_index("s")
        base = (cid * NS + sid) * EPT

        bufs = ((ba0, bb0, bo0, br0, is0, id0, sa0, sb0, so0, si0),
                (ba1, bb1, bo1, br1, is1, id1, sa1, sb1, so1, si1))

        def idx_copy(j, p):
            isb, idb, si = bufs[p][4], bufs[p][5], bufs[p][9]
            pltpu.async_copy(src_h.at[cid, sid, j], isb, si)
            pltpu.async_copy(dst_h.at[cid, sid, j], idb, si)

        def issue(j, p):
            ba, bb, _, _, isb, idb, sa, sb, _, si = bufs[p]
            pltpu.make_async_copy(src_h.at[cid, sid, j], isb, si).wait()
            pltpu.make_async_copy(dst_h.at[cid, sid, j], idb, si).wait()
            pltpu.async_copy(ta_h.at[isb.at[0]], ba, sa)
            pltpu.async_copy(tb_h.at[idb.at[0]], bb, sb)

        def process(j, p):
            ba, bb, bo, br, isb, idb, sa, sb, so, si = bufs[p]
            idx_copy(j + 2, p)

            pltpu.make_async_copy(ta_h.at[isb.at[0]], ba, sa).wait()
            pltpu.make_async_copy(tb_h.at[idb.at[0]], bb, sb).wait()

            pltpu.make_async_copy(bo, o_h.at[pl.ds(base, CHK)], so).wait()
            pltpu.make_async_copy(br, r_h.at[pl.ds(base, CHK)], so).wait()

            def add_row(r4, carry):
                for u in range(4):
                    r = r4 * 4 + u
                    for c in range(D // 16):
                        sl = pl.ds(c * 16, 16)
                        bo[r, sl] = ba[r, sl] + bb[r, sl]
                    sl = pl.ds(D, 16)
                    br[r, pl.ds(0, 16)] = ba[r, sl] + bb[r, sl]
                return carry

            lax.fori_loop(0, CHK // 4, add_row, 0)

            pltpu.async_copy(bo, o_h.at[pl.ds(base + j * CHK, CHK)], so)
            pltpu.async_copy(br, r_h.at[pl.ds(base + j * CHK, CHK)], so)

        idx_copy(0, 0)
        idx_copy(1, 1)
        issue(0, 0)
        issue(1, 1)
        pltpu.async_copy(bo0, o_h.at[pl.ds(base, CHK)], so0)
        pltpu.async_copy(br0, r_h.at[pl.ds(base, CHK)], so0)
        pltpu.async_copy(bo1, o_h.at[pl.ds(base, CHK)], so1)
        pltpu.async_copy(br1, r_h.at[pl.ds(base, CHK)], so1)

        def body(j2, carry):
            j = 2 * j2
            process(j, 0)
            issue(j + 2, 0)
            process(j + 1, 1)
            issue(j + 3, 1)
            return carry

        lax.fori_loop(0, (CH - 1) // 2, body, 0)
        process(CH - 1, 0)
        pltpu.make_async_copy(bo1, o_h.at[pl.ds(base, CHK)], so1).wait()
        pltpu.make_async_copy(br1, r_h.at[pl.ds(base, CHK)], so1).wait()
        pltpu.make_async_copy(bo0, o_h.at[pl.ds(base, CHK)], so0).wait()
        pltpu.make_async_copy(br0, r_h.at[pl.ds(base, CHK)], so0).wait()
        pltpu.make_async_copy(ta_h.at[is1.at[0]], ba1, sa1).wait()
        pltpu.make_async_copy(tb_h.at[id1.at[0]], bb1, sb1).wait()
        pltpu.make_async_copy(src_h.at[cid, sid, 0], is0, si0).wait()
        pltpu.make_async_copy(dst_h.at[cid, sid, 0], id0, si0).wait()

    return k(ta, tb, src5, dst5)


def _edge_body(o_ref, r_ref, ea_ref, we_ref, wd_ref, bm1_ref, wm2_ref,
               bm2_ref, wc1_ref, bc1_ref, wc2_ref, bc2_ref, sm_ref, sc_ref):
    g = o_ref[...]
    r16 = r_ref[...]
    ds = jnp.sum(r16 * r16, axis=1, keepdims=True)
    mp = (g + jnp.dot(ea_ref[...], we_ref[...],
                      preferred_element_type=jnp.float32)
          + ds * wd_ref[...] + bm1_ref[...])
    m = _silu(mp)
    bf = jnp.bfloat16
    msg = _silu(jnp.dot(m.astype(bf), wm2_ref[...].astype(bf),
                        preferred_element_type=jnp.float32) + bm2_ref[...])
    t = _silu(jnp.dot(msg.astype(bf), wc1_ref[...].astype(bf),
                      preferred_element_type=jnp.float32) + bc1_ref[...])
    cw = jnp.dot(t.astype(bf), wc2_ref[...].astype(bf),
                 preferred_element_type=jnp.float32) + bc2_ref[...]
    sm_ref[...] = msg
    cs16 = r16 * cw
    col = lax.broadcasted_iota(jnp.int32, cs16.shape, 1)
    cs16 = jnp.where(col == 3, 1.0, cs16)
    sc_ref[...] = jnp.concatenate(
        [cs16, jnp.zeros((cs16.shape[0], D - 16), jnp.float32)], axis=1)


def _edge(o, r, ea, we, wd, bm1, wm2, bm2, wc1, bc1, wc2, bc2):
    grid = (E // BE,)
    full = lambda i: (0, 0)
    return pl.pallas_call(
        _edge_body,
        grid=grid,
        in_specs=[
            pl.BlockSpec((BE, D), lambda i: (i, 0)),
            pl.BlockSpec((BE, 16), lambda i: (i, 0)),
            pl.BlockSpec((BE, ED), lambda i: (i, 0)),
            pl.BlockSpec((ED, H), full),
            pl.BlockSpec((1, H), full),
            pl.BlockSpec((1, H), full),
            pl.BlockSpec((H, H), full),
            pl.BlockSpec((1, H), full),
            pl.BlockSpec((H, H), full),
            pl.BlockSpec((1, H), full),
            pl.BlockSpec((H, 1), full),
            pl.BlockSpec((1, 1), full),
        ],
        out_specs=[
            pl.BlockSpec((BE, D), lambda i: (i, 0)),
            pl.BlockSpec((BE, D), lambda i: (i, 0)),
        ],
        out_shape=[
            jax.ShapeDtypeStruct((E, D), jnp.float32),
            jax.ShapeDtypeStruct((E, D), jnp.float32),
        ],
    )(o, r, ea, we, wd, bm1, wm2, bm2, wc1, bc1, wc2, bc2)


def _scatter(sm, sc_, dst3, zer):
    @functools.partial(
        pl.kernel,
        out_type=jax.ShapeDtypeStruct((NC, N, D), jnp.float32),
        mesh=_sc_mesh(),
        scratch_types=[
            pltpu.VMEM((1, CHK), jnp.int32),
            pltpu.VMEM((1, CHK), jnp.int32),
            pltpu.VMEM((CHK, D), jnp.float32),
            pltpu.VMEM((CHK, D), jnp.float32),
            pltpu.VMEM_SHARED((N, D), jnp.float32),
            pltpu.SemaphoreType.DMA,
            pltpu.SemaphoreType.DMA,
            pltpu.SemaphoreType.DMA,
            pltpu.SemaphoreType.DMA,
        ],
    )
    def k(sm_h, sc_h, dst_h, z_h, p_h, ix0, ix1, pay0, pay1, accum,
          l0, l1, s0, s1):
        cid = lax.axis_index("c")
        sid = lax.axis_index("s")

        @pl.when(sid == 0)
        def _():
            pltpu.sync_copy(z_h, accum)

        plsc.subcore_barrier()
        base = sid * (CH2 * CHK)

        def run(src_h):
            def load(j, pay, ixb, sem):
                pltpu.async_copy(
                    src_h.at[pl.ds(base + j * CHK, CHK)], pay, sem)
                pltpu.async_copy(dst_h.at[sid, j], ixb, sem)

            def wait_load(pay, ixb, sem):
                pltpu.make_async_copy(
                    src_h.at[pl.ds(base, CHK)], pay, sem).wait()
                pltpu.make_async_copy(dst_h.at[sid, 0], ixb, sem).wait()

            def scat(pay, ixb, sem):
                pltpu.async_copy(pay, accum.at[ixb.at[0]], sem, add=True)

            def wait_scat(pay, ixb, sem):
                pltpu.make_async_copy(
                    pay, accum.at[ixb.at[0]], sem).wait()

            load(0, pay0, ix0, l0)

            def body(j2, carry):
                j = 2 * j2
                wait_load(pay0, ix0, l0)
                scat(pay0, ix0, s0)

                @pl.when(j2 > 0)
                def _():
                    wait_scat(pay1, ix1, s1)

                load(j + 1, pay1, ix1, l1)
                wait_load(pay1, ix1, l1)
                scat(pay1, ix1, s1)
                wait_scat(pay0, ix0, s0)

                @pl.when(j2 < CH2 // 2 - 1)
                def _():
                    load(j + 2, pay0, ix0, l0)

                return carry

            lax.fori_loop(0, CH2 // 2, body, 0)
            wait_scat(pay1, ix1, s1)

        @pl.when(cid == 0)
        def _():
            run(sm_h)

        @pl.when(cid == 1)
        def _():
            run(sc_h)

        plsc.subcore_barrier()
        rows = 1000

        @pl.when(sid < N // rows)
        def _():
            pltpu.sync_copy(accum.at[pl.ds(sid * rows, rows)],
                            p_h.at[cid, pl.ds(sid * rows, rows)])

    return k(sm, sc_, dst3, zer)


def _node_body(h_ref, xp_ref, p_ref, wn1a_ref, wn1b_ref, bn1_ref,
               wn2_ref, bn2_ref, g_ref, b_ref, ho_ref, xo_ref):
    h = h_ref[...]
    p = p_ref[...]
    am = p[0]
    a16 = p[1, :, :16]
    cnt = jnp.maximum(a16[:, 3:4], 1.0)
    xo_ref[...] = xp_ref[...] + a16 / cnt
    hu = _silu(jnp.dot(h, wn1a_ref[...], preferred_element_type=jnp.float32)
               + jnp.dot(am, wn1b_ref[...], preferred_element_type=jnp.float32)
               + bn1_ref[...])
    hu = jnp.dot(hu, wn2_ref[...],
                 preferred_element_type=jnp.float32) + bn2_ref[...]
    pre = h + hu
    mu = jnp.mean(pre, axis=1, keepdims=True)
    c = pre - mu
    var = jnp.mean(c * c, axis=1, keepdims=True)
    ho_ref[...] = c * lax.rsqrt(var + 1e-5) * g_ref[...] + b_ref[...]


def _node(h, xp, parts, wn1a, wn1b, bn1, wn2, bn2, gamma, beta):
    grid = (N // BN,)
    full = lambda i: (0, 0)
    return pl.pallas_call(
        _node_body,
        grid=grid,
        in_specs=[
            pl.BlockSpec((BN, D), lambda i: (i, 0)),
            pl.BlockSpec((BN, 16), lambda i: (i, 0)),
            pl.BlockSpec((2, BN, D), lambda i: (0, i, 0)),
            pl.BlockSpec((D, H), full),
            pl.BlockSpec((H, H), full),
            pl.BlockSpec((1, H), full),
            pl.BlockSpec((H, D), full),
            pl.BlockSpec((1, D), full),
            pl.BlockSpec((1, D), full),
            pl.BlockSpec((1, D), full),
        ],
        out_specs=[
            pl.BlockSpec((BN, D), lambda i: (i, 0)),
            pl.BlockSpec((BN, 16), lambda i: (i, 0)),
        ],
        out_shape=[
            jax.ShapeDtypeStruct((N, D), jnp.float32),
            jax.ShapeDtypeStruct((N, 16), jnp.float32),
        ],
    )(h, xp, parts, wn1a, wn1b, bn1, wn2, bn2, gamma, beta)


def kernel(h, x, edge_index, edge_attr, Wm1, bm1, Wm2, bm2, Wc1, bc1, Wc2,
           bc2, Wn1, bn1, Wn2, bn2, gamma, beta):
    pad2 = ((0, 0), (0, 0), (0, 2), (0, 0), (0, 0))
    src5 = jnp.pad(edge_index[0].reshape(NC, NS, CH, 1, CHK), pad2)
    dst5 = jnp.pad(edge_index[1].reshape(NC, NS, CH, 1, CHK), pad2)
    dst3 = edge_index[1].reshape(NS, CH2, 1, CHK)
    xp = jnp.pad(x, ((0, 0), (0, 13)))
    xn = -xp

    ta, tb = _prep(h, xp, xn, Wm1[:D], Wm1[D:2 * D])
    o, r = _gather(ta, tb, src5, dst5)

    sm, sc_ = _edge(o, r, edge_attr, Wm1[2 * D:2 * D + ED], Wm1[2 * D + ED:],
                    bm1[None, :], Wm2, bm2[None, :], Wc1, bc1[None, :], Wc2,
                    bc2[None, :])

    parts = _scatter(sm, sc_, dst3, jnp.zeros((N, D), jnp.float32))

    ho, xo = _node(h, xp, parts, Wn1[:D], Wn1[D:], bn1[None, :], Wn2,
                   bn2[None, :], gamma[None, :], beta[None, :])
    return (ho, xo[:, :3])


# speedup vs baseline: 1.0149x; 1.0149x over previous
import functools

import jax
import jax.numpy as jnp
from jax import lax
from jax.experimental import pallas as pl
from jax.experimental.pallas import tpu as pltpu
from jax.experimental.pallas import tpu_sc as plsc

N = 10000
E = 320000
D = 128
ED = 16
H = 128
W = 144

BN = 1000
BE = 2000

NC = 2
NS = 16
CHK = 80
CH = E // (NC * NS * CHK)
EPT = CH * CHK
CH2 = E // (NS * CHK)


def _silu(v):
    return v * jax.nn.sigmoid(v)


WG = 256


def _prep_body(h_ref, xp_ref, xn_ref, wa_ref, wb_ref, ts_ref, td_ref):
    h = h_ref[...]
    z = jnp.zeros((h.shape[0], WG - D - 16), jnp.float32)
    ts_ref[...] = jnp.concatenate([jnp.dot(h, wa_ref[...],
                                           preferred_element_type=jnp.float32),
                                   xp_ref[...], z], axis=1)
    td_ref[...] = jnp.concatenate([jnp.dot(h, wb_ref[...],
                                           preferred_element_type=jnp.float32),
                                   xn_ref[...], z], axis=1)


def _prep(h, xp, xn, wa, wb):
    grid = (N // BN,)
    return pl.pallas_call(
        _prep_body,
        grid=grid,
        in_specs=[
            pl.BlockSpec((BN, D), lambda i: (i, 0)),
            pl.BlockSpec((BN, 16), lambda i: (i, 0)),
            pl.BlockSpec((BN, 16), lambda i: (i, 0)),
            pl.BlockSpec((D, D), lambda i: (0, 0)),
            pl.BlockSpec((D, D), lambda i: (0, 0)),
        ],
        out_specs=[
            pl.BlockSpec((BN, WG), lambda i: (i, 0)),
            pl.BlockSpec((BN, WG), lambda i: (i, 0)),
        ],
        out_shape=[
            jax.ShapeDtypeStruct((N, WG), jnp.float32),
            jax.ShapeDtypeStruct((N, WG), jnp.float32),
        ],
    )(h, xp, xn, wa, wb)


def _sc_mesh():
    return plsc.VectorSubcoreMesh(core_axis_name="c", subcore_axis_name="s")


def _gather(ta, tb, src5, dst5):
    @functools.partial(
        pl.kernel,
        out_type=[
            jax.ShapeDtypeStruct((E, D), jnp.float32),
            jax.ShapeDtypeStruct((E, 16), jnp.float32),
        ],
        mesh=_sc_mesh(),
        scratch_types=[
            pltpu.VMEM((1, CHK), jnp.int32),
            pltpu.VMEM((1, CHK), jnp.int32),
            pltpu.VMEM((1, CHK), jnp.int32),
            pltpu.VMEM((1, CHK), jnp.int32),
            pltpu.VMEM((CHK, WG), jnp.float32),
            pltpu.VMEM((CHK, WG), jnp.float32),
            pltpu.VMEM((CHK, WG), jnp.float32),
            pltpu.VMEM((CHK, WG), jnp.float32),
            pltpu.VMEM((CHK, D), jnp.float32),
            pltpu.VMEM((CHK, D), jnp.float32),
            pltpu.VMEM((CHK, 16), jnp.float32),
            pltpu.VMEM((CHK, 16), jnp.float32),
            pltpu.SemaphoreType.DMA,
            pltpu.SemaphoreType.DMA,
            pltpu.SemaphoreType.DMA,
            pltpu.SemaphoreType.DMA,
            pltpu.SemaphoreType.DMA,
            pltpu.SemaphoreType.DMA,
            pltpu.SemaphoreType.DMA,
            pltpu.SemaphoreType.DMA,
        ],
    )
    def k(ta_h, tb_h, src_h, dst_h, o_h, r_h,
          is0, is1, id0, id1,
          ba0, ba1, bb0, bb1, bo0, bo1, br0, br1,
          sa0, sa1, sb0, sb1, so0, so1, si0, si1):
        cid = lax.axis_index("c")
        sid = lax.axis_index("s")
        base = (cid * NS + sid) * EPT

        bufs = ((ba0, bb0, bo0, br0, is0, id0, sa0, sb0, so0, si0),
                (ba1, bb1, bo1, br1, is1, id1, sa1, sb1, so1, si1))

        def idx_copy(j, p):
            isb, idb, si = bufs[p][4], bufs[p][5], bufs[p][9]
            pltpu.async_copy(src_h.at[cid, sid, j], isb, si)
            pltpu.async_copy(dst_h.at[cid, sid, j], idb, si)

        def issue(j, p):
            ba, bb, _, _, isb, idb, sa, sb, _, si = bufs[p]
            pltpu.make_async_copy(src_h.at[cid, sid, j], isb, si).wait()
            pltpu.make_async_copy(dst_h.at[cid, sid, j], idb, si).wait()
            pltpu.async_copy(ta_h.at[isb.at[0]], ba, sa)
            pltpu.async_copy(tb_h.at[idb.at[0]], bb, sb)

        def process(j, p):
            ba, bb, bo, br, isb, idb, sa, sb, so, si = bufs[p]
            idx_copy(j + 2, p)

            pltpu.make_async_copy(ta_h.at[isb.at[0]], ba, sa).wait()
            pltpu.make_async_copy(tb_h.at[idb.at[0]], bb, sb).wait()

            pltpu.make_async_copy(bo, o_h.at[pl.ds(base, CHK)], so).wait()
            pltpu.make_async_copy(br, r_h.at[pl.ds(base, CHK)], so).wait()

            def add_row(r4, carry):
                for u in range(4):
                    r = r4 * 4 + u
                    for c in range(D // 16):
                        sl = pl.ds(c * 16, 16)
                        bo[r, sl] = ba[r, sl] + bb[r, sl]
                    sl = pl.ds(D, 16)
                    br[r, pl.ds(0, 16)] = ba[r, sl] + bb[r, sl]
                return carry

            lax.fori_loop(0, CHK // 4, add_row, 0)

            pltpu.async_copy(bo, o_h.at[pl.ds(base + j * CHK, CHK)], so)
            pltpu.async_copy(br, r_h.at[pl.ds(base + j * CHK, CHK)], so)

        idx_copy(0, 0)
        idx_copy(1, 1)
        issue(0, 0)
        issue(1, 1)
        pltpu.async_copy(bo0, o_h.at[pl.ds(base, CHK)], so0)
        pltpu.async_copy(br0, r_h.at[pl.ds(base, CHK)], so0)
        pltpu.async_copy(bo1, o_h.at[pl.ds(base, CHK)], so1)
        pltpu.async_copy(br1, r_h.at[pl.ds(base, CHK)], so1)

        def body(j2, carry):
            j = 2 * j2
            process(j, 0)
            issue(j + 2, 0)
            process(j + 1, 1)
            issue(j + 3, 1)
            return carry

        lax.fori_loop(0, (CH - 1) // 2, body, 0)
        process(CH - 1, 0)
        pltpu.make_async_copy(bo1, o_h.at[pl.ds(base, CHK)], so1).wait()
        pltpu.make_async_copy(br1, r_h.at[pl.ds(base, CHK)], so1).wait()
        pltpu.make_async_copy(bo0, o_h.at[pl.ds(base, CHK)], so0).wait()
        pltpu.make_async_copy(br0, r_h.at[pl.ds(base, CHK)], so0).wait()
        pltpu.make_async_copy(ta_h.at[is1.at[0]], ba1, sa1).wait()
        pltpu.make_async_copy(tb_h.at[id1.at[0]], bb1, sb1).wait()
        pltpu.make_async_copy(src_h.at[cid, sid, 0], is0, si0).wait()
        pltpu.make_async_copy(dst_h.at[cid, sid, 0], id0, si0).wait()

    return k(ta, tb, src5, dst5)


def _edge_body(o_ref, r_ref, ea_ref, we_ref, wd_ref, bm1_ref, wm2_ref,
               bm2_ref, wc1_ref, bc1_ref, wc2_ref, bc2_ref, sm_ref, sc_ref):
    g = o_ref[...]
    r16 = r_ref[...]
    ds = jnp.sum(r16 * r16, axis=1, keepdims=True)
    mp = (g + jnp.dot(ea_ref[...], we_ref[...],
                      preferred_element_type=jnp.float32)
          + ds * wd_ref[...] + bm1_ref[...])
    m = _silu(mp)
    msg = _silu(jnp.dot(m, wm2_ref[...],
                        preferred_element_type=jnp.float32) + bm2_ref[...])
    t = _silu(jnp.dot(msg, wc1_ref[...],
                      preferred_element_type=jnp.float32) + bc1_ref[...])
    cw = jnp.dot(t, wc2_ref[...],
                 preferred_element_type=jnp.float32) + bc2_ref[...]
    sm_ref[...] = msg
    cs16 = r16 * cw
    col = lax.broadcasted_iota(jnp.int32, cs16.shape, 1)
    cs16 = jnp.where(col == 3, 1.0, cs16)
    sc_ref[...] = jnp.concatenate(
        [cs16, jnp.zeros((cs16.shape[0], D - 16), jnp.float32)], axis=1)


def _edge(o, r, ea, we, wd, bm1, wm2, bm2, wc1, bc1, wc2, bc2):
    grid = (E // BE,)
    full = lambda i: (0, 0)
    return pl.pallas_call(
        _edge_body,
        grid=grid,
        in_specs=[
            pl.BlockSpec((BE, D), lambda i: (i, 0)),
            pl.BlockSpec((BE, 16), lambda i: (i, 0)),
            pl.BlockSpec((BE, ED), lambda i: (i, 0)),
            pl.BlockSpec((ED, H), full),
            pl.BlockSpec((1, H), full),
            pl.BlockSpec((1, H), full),
            pl.BlockSpec((H, H), full),
            pl.BlockSpec((1, H), full),
            pl.BlockSpec((H, H), full),
            pl.BlockSpec((1, H), full),
            pl.BlockSpec((H, 1), full),
            pl.BlockSpec((1, 1), full),
        ],
        out_specs=[
            pl.BlockSpec((BE, D), lambda i: (i, 0)),
            pl.BlockSpec((BE, D), lambda i: (i, 0)),
        ],
        out_shape=[
            jax.ShapeDtypeStruct((E, D), jnp.float32),
            jax.ShapeDtypeStruct((E, D), jnp.float32),
        ],
    )(o, r, ea, we, wd, bm1, wm2, bm2, wc1, bc1, wc2, bc2)


def _scatter(sm, sc_, dst3, zer):
    @functools.partial(
        pl.kernel,
        out_type=jax.ShapeDtypeStruct((NC, N, D), jnp.float32),
        mesh=_sc_mesh(),
        scratch_types=[
            pltpu.VMEM((1, CHK), jnp.int32),
            pltpu.VMEM((1, CHK), jnp.int32),
            pltpu.VMEM((CHK, D), jnp.float32),
            pltpu.VMEM((CHK, D), jnp.float32),
            pltpu.VMEM_SHARED((N, D), jnp.float32),
            pltpu.SemaphoreType.DMA,
            pltpu.SemaphoreType.DMA,
            pltpu.SemaphoreType.DMA,
            pltpu.SemaphoreType.DMA,
        ],
    )
    def k(sm_h, sc_h, dst_h, z_h, p_h, ix0, ix1, pay0, pay1, accum,
          l0, l1, s0, s1):
        cid = lax.axis_index("c")
        sid = lax.axis_index("s")

        @pl.when(sid == 0)
        def _():
            pltpu.sync_copy(z_h, accum)

        plsc.subcore_barrier()
        base = sid * (CH2 * CHK)

        def run(src_h):
            def load(j, pay, ixb, sem):
                pltpu.async_copy(
                    src_h.at[pl.ds(base + j * CHK, CHK)], pay, sem)
                pltpu.async_copy(dst_h.at[sid, j], ixb, sem)

            def wait_load(pay, ixb, sem):
                pltpu.make_async_copy(
                    src_h.at[pl.ds(base, CHK)], pay, sem).wait()
                pltpu.make_async_copy(dst_h.at[sid, 0], ixb, sem).wait()

            def scat(pay, ixb, sem):
                pltpu.async_copy(pay, accum.at[ixb.at[0]], sem, add=True)

            def wait_scat(pay, ixb, sem):
                pltpu.make_async_copy(
                    pay, accum.at[ixb.at[0]], sem).wait()

            load(0, pay0, ix0, l0)

            def body(j2, carry):
                j = 2 * j2
                wait_load(pay0, ix0, l0)
                scat(pay0, ix0, s0)

                @pl.when(j2 > 0)
                def _():
                    wait_scat(pay1, ix1, s1)

                load(j + 1, pay1, ix1, l1)
                wait_load(pay1, ix1, l1)
                scat(pay1, ix1, s1)
                wait_scat(pay0, ix0, s0)

                @pl.when(j2 < CH2 // 2 - 1)
                def _():
                    load(j + 2, pay0, ix0, l0)

                return carry

            lax.fori_loop(0, CH2 // 2, body, 0)
            wait_scat(pay1, ix1, s1)

        @pl.when(cid == 0)
        def _():
            run(sm_h)

        @pl.when(cid == 1)
        def _():
            run(sc_h)

        plsc.subcore_barrier()
        rows = 1000

        @pl.when(sid < N // rows)
        def _():
            pltpu.sync_copy(accum.at[pl.ds(sid * rows, rows)],
                            p_h.at[cid, pl.ds(sid * rows, rows)])

    return k(sm, sc_, dst3, zer)


def _node_body(h_ref, xp_ref, p_ref, wn1a_ref, wn1b_ref, bn1_ref,
               wn2_ref, bn2_ref, g_ref, b_ref, ho_ref, xo_ref):
    h = h_ref[...]
    p = p_ref[...]
    am = p[0]
    a16 = p[1, :, :16]
    cnt = jnp.maximum(a16[:, 3:4], 1.0)
    xo_ref[...] = xp_ref[...] + a16 / cnt
    hu = _silu(jnp.dot(h, wn1a_ref[...], preferred_element_type=jnp.float32)
               + jnp.dot(am, wn1b_ref[...], preferred_element_type=jnp.float32)
               + bn1_ref[...])
    hu = jnp.dot(hu, wn2_ref[...],
                 preferred_element_type=jnp.float32) + bn2_ref[...]
    pre = h + hu
    mu = jnp.mean(pre, axis=1, keepdims=True)
    c = pre - mu
    var = jnp.mean(c * c, axis=1, keepdims=True)
    ho_ref[...] = c * lax.rsqrt(var + 1e-5) * g_ref[...] + b_ref[...]


def _node(h, xp, parts, wn1a, wn1b, bn1, wn2, bn2, gamma, beta):
    grid = (N // BN,)
    full = lambda i: (0, 0)
    return pl.pallas_call(
        _node_body,
        grid=grid,
        in_specs=[
            pl.BlockSpec((BN, D), lambda i: (i, 0)),
            pl.BlockSpec((BN, 16), lambda i: (i, 0)),
            pl.BlockSpec((2, BN, D), lambda i: (0, i, 0)),
            pl.BlockSpec((D, H), full),
            pl.BlockSpec((H, H), full),
            pl.BlockSpec((1, H), full),
            pl.BlockSpec((H, D), full),
            pl.BlockSpec((1, D), full),
            pl.BlockSpec((1, D), full),
            pl.BlockSpec((1, D), full),
        ],
        out_specs=[
            pl.BlockSpec((BN, D), lambda i: (i, 0)),
            pl.BlockSpec((BN, 16), lambda i: (i, 0)),
        ],
        out_shape=[
            jax.ShapeDtypeStruct((N, D), jnp.float32),
            jax.ShapeDtypeStruct((N, 16), jnp.float32),
        ],
    )(h, xp, parts, wn1a, wn1b, bn1, wn2, bn2, gamma, beta)


def kernel(h, x, edge_index, edge_attr, Wm1, bm1, Wm2, bm2, Wc1, bc1, Wc2,
           bc2, Wn1, bn1, Wn2, bn2, gamma, beta):
    pad2 = ((0, 0), (0, 0), (0, 2), (0, 0), (0, 0))
    src5 = jnp.pad(edge_index[0].reshape(NC, NS, CH, 1, CHK), pad2)
    dst5 = jnp.pad(edge_index[1].reshape(NC, NS, CH, 1, CHK), pad2)
    dst3 = edge_index[1].reshape(NS, CH2, 1, CHK)
    xp = jnp.pad(x, ((0, 0), (0, 13)))
    xn = -xp

    ta, tb = _prep(h, xp, xn, Wm1[:D], Wm1[D:2 * D])
    o, r = _gather(ta, tb, src5, dst5)

    sm, sc_ = _edge(o, r, edge_attr, Wm1[2 * D:2 * D + ED], Wm1[2 * D + ED:],
                    bm1[None, :], Wm2, bm2[None, :], Wc1, bc1[None, :], Wc2,
                    bc2[None, :])

    parts = _scatter(sm, sc_, dst3, jnp.zeros((N, D), jnp.float32))

    ho, xo = _node(h, xp, parts, Wn1[:D], Wn1[D:], bn1[None, :], Wn2,
                   bn2[None, :], gamma[None, :], beta[None, :])
    return (ho, xo[:, :3])
